# baseline (speedup 1.0000x reference)
import functools
import math

import jax
import jax.numpy as jnp
from jax import lax
from jax.experimental import pallas as pl
from jax.experimental.pallas import tpu as pltpu
from jax.experimental.pallas import tpu_sc as plsc

_B, _NF, _D, _FF, _E, _H = 1024, 20, 256, 512, 5, 8
_S = _NF + 1
_T = _B * _S
_DH = _D // _H
_TILE = 512
_NT = _T // _TILE
_G1 = 8
_G2 = 16
_NEG = -1e30

_NC, _NS = 2, 16
_NW = _NC * _NS
_CHUNK = _T // _NW
_NVR = _CHUNK // 16
_CC = 112
_NCH = _CHUNK // _CC
_VPC = _CC // 16


def _f32dot(a, b):
    return jnp.dot(a.astype(jnp.bfloat16), b.astype(jnp.bfloat16),
                   preferred_element_type=jnp.float32)


def _bdg(a, b, dims):
    return lax.dot_general(a.astype(jnp.bfloat16), b.astype(jnp.bfloat16),
                           dims, preferred_element_type=jnp.float32)


def _ln(x, g, b):
    m = jnp.mean(x, axis=-1, keepdims=True)
    v = jnp.mean((x - m) ** 2, axis=-1, keepdims=True)
    return (x - m) / jnp.sqrt(v + 1e-5) * g + b


def _full_spec(shape):
    n = len(shape)
    return pl.BlockSpec(shape, lambda i, _n=n: (0,) * _n)


def _div21(x):
    return (x * 3121) >> 16


def _embed_body(x_ref, w_ref, b_ref, cls_ref, g_ref, bb_ref, o_ref):
    x = x_ref[...]
    w = w_ref[...][None]
    b = b_ref[...][None]
    feat = x[:, :, None] * w + b
    cls = jnp.broadcast_to(cls_ref[...][None], (x.shape[0], 1, _D))
    h = jnp.concatenate([cls, feat], axis=1)
    o_ref[...] = _ln(h, g_ref[...][None], bb_ref[...][None])


def _embed(x, p):
    bb = 256
    return pl.pallas_call(
        _embed_body,
        grid=(_B // bb,),
        in_specs=[
            pl.BlockSpec((bb, _NF), lambda i: (i, 0)),
            _full_spec((1, _D)), _full_spec((1, _D)), _full_spec((1, _D)),
            _full_spec((1, _D)), _full_spec((1, _D)),
        ],
        out_specs=pl.BlockSpec((bb, _S, _D), lambda i: (i, 0, 0)),
        out_shape=jax.ShapeDtypeStruct((_B, _S, _D), jnp.float32),
    )(x, p['con_w'].reshape(1, _D), p['con_b'].reshape(1, _D),
      p['cls'].reshape(1, _D), p['norm_g'].reshape(1, _D),
      p['norm_b'].reshape(1, _D))


def _qkv_body(h_ref, g_ref, b_ref, w_ref, bias_ref, o_ref):
    u = _ln(h_ref[...], g_ref[...], b_ref[...])
    o_ref[...] = _f32dot(u, w_ref[...]) + bias_ref[...]


def _qkv(h, bp):
    return pl.pallas_call(
        _qkv_body,
        grid=(_NT,),
        in_specs=[
            pl.BlockSpec((_TILE, _D), lambda i: (i, 0)),
            _full_spec((1, _D)), _full_spec((1, _D)),
            _full_spec((_D, 3 * _D)), _full_spec((1, 3 * _D)),
        ],
        out_specs=pl.BlockSpec((_TILE, 3 * _D), lambda i: (i, 0)),
        out_shape=jax.ShapeDtypeStruct((_T, 3 * _D), jnp.float32),
    )(h, bp['ln1_g'].reshape(1, _D), bp['ln1_b'].reshape(1, _D),
      bp['wqkv'], bp['bqkv'].reshape(1, 3 * _D))


def _split_heads(t):
    return jnp.concatenate([t[:, :, _DH * h:_DH * (h + 1)] for h in range(_H)],
                           axis=1)


def _attn1_body(qkv_ref, o_ref):
    u = qkv_ref[...]
    qh = _split_heads(u[:, :, :_D])
    kh = _split_heads(u[:, :, _D:2 * _D])
    vh = _split_heads(u[:, :, 2 * _D:])
    s = _bdg(qh, kh, (((2,), (2,)), ((0,), (0,))))
    hs = _H * _S
    rh = _div21(lax.broadcasted_iota(jnp.int32, (hs, hs), 0))
    ch = _div21(lax.broadcasted_iota(jnp.int32, (hs, hs), 1))
    s = jnp.where((rh == ch)[None], s * (1.0 / math.sqrt(_DH)), _NEG)
    s = s - jnp.max(s, axis=-1, keepdims=True)
    e = jnp.exp(s)
    a = e / jnp.sum(e, axis=-1, keepdims=True)
    o = _bdg(a, vh, (((2,), (1,)), ((0,), (0,))))
    o_ref[...] = jnp.concatenate(
        [o[:, _S * h:_S * (h + 1), :] for h in range(_H)], axis=2)


def _attn1(qkv):
    return pl.pallas_call(
        _attn1_body,
        grid=(_B // _G1,),
        in_specs=[pl.BlockSpec((_G1, _S, 3 * _D), lambda i: (i, 0, 0))],
        out_specs=pl.BlockSpec((_G1, _S, _D), lambda i: (i, 0, 0)),
        out_shape=jax.ShapeDtypeStruct((_B, _S, _D), jnp.float32),
    )(qkv)


def _attn2_body(qkv_ref, o_ref):
    u = qkv_ref[...]
    q0 = u[:, 0:1, :_D]
    qh = _split_heads(q0)
    kh = _split_heads(u[:, :, _D:2 * _D])
    vh = _split_heads(u[:, :, 2 * _D:])
    s = _bdg(qh, kh, (((2,), (2,)), ((0,), (0,))))
    hs = _H * _S
    rh = lax.broadcasted_iota(jnp.int32, (_H, hs), 0)
    ch = _div21(lax.broadcasted_iota(jnp.int32, (_H, hs), 1))
    s = jnp.where((rh == ch)[None], s * (1.0 / math.sqrt(_DH)), _NEG)
    s = s - jnp.max(s, axis=-1, keepdims=True)
    e = jnp.exp(s)
    a = e / jnp.sum(e, axis=-1, keepdims=True)
    o = _bdg(a, vh, (((2,), (1,)), ((0,), (0,))))
    o_ref[...] = jnp.concatenate(
        [o[:, h:h + 1, :] for h in range(_H)], axis=2)


def _attn2(qkv):
    return pl.pallas_call(
        _attn2_body,
        grid=(_B // _G2,),
        in_specs=[pl.BlockSpec((_G2, _S, 3 * _D), lambda i: (i, 0, 0))],
        out_specs=pl.BlockSpec((_G2, 1, _D), lambda i: (i, 0, 0)),
        out_shape=jax.ShapeDtypeStruct((_B, 1, _D), jnp.float32),
    )(qkv)


def _proj_body(h_ref, a_ref, w_ref, b_ref, o_ref):
    o_ref[...] = h_ref[...] + _f32dot(a_ref[...], w_ref[...]) + b_ref[...]


def _proj(h, att, bp):
    return pl.pallas_call(
        _proj_body,
        grid=(_NT,),
        in_specs=[
            pl.BlockSpec((_TILE, _D), lambda i: (i, 0)),
            pl.BlockSpec((_TILE, _D), lambda i: (i, 0)),
            _full_spec((_D, _D)), _full_spec((1, _D)),
        ],
        out_specs=pl.BlockSpec((_TILE, _D), lambda i: (i, 0)),
        out_shape=jax.ShapeDtypeStruct((_T, _D), jnp.float32),
    )(h, att, bp['wo'], bp['bo'].reshape(1, _D))


def _moe_math(h, g2, b2, cent, w1, b1, w2, b2e, ws1, bs1, ws2, bs2):
    u2 = _ln(h, g2, b2)
    logits = _f32dot(u2, cent)
    mx = jnp.max(logits, axis=-1, keepdims=True)
    eg = jnp.exp(logits - mx)
    gate = eg / jnp.sum(eg, axis=-1, keepdims=True)
    iot = lax.broadcasted_iota(jnp.int32, logits.shape, 1)
    top = jnp.min(jnp.where(logits == mx, iot, _E), axis=-1, keepdims=True)
    acc = jnp.zeros_like(h)
    for e in range(_E):
        t1 = jax.nn.gelu(_f32dot(u2, w1[e]) + b1[e][None])
        t2 = _f32dot(t1, w2[e]) + b2e[e][None]
        sel = jnp.where(top == e, gate[:, e:e + 1], 0.0)
        acc = acc + sel * t2
    ys = _f32dot(jax.nn.gelu(_f32dot(u2, ws1) + bs1), ws2) + bs2
    return h + acc + ys


def _moe_body(h_ref, g2_ref, b2_ref, cent_ref, w1_ref, b1_ref, w2_ref,
              b2e_ref, ws1_ref, bs1_ref, ws2_ref, bs2_ref, o_ref):
    o_ref[...] = _moe_math(
        h_ref[...], g2_ref[...], b2_ref[...], cent_ref[...], w1_ref[...],
        b1_ref[...], w2_ref[...], b2e_ref[...], ws1_ref[...], bs1_ref[...],
        ws2_ref[...], bs2_ref[...])


def _moe_dense(h, bp, cent_t):
    return pl.pallas_call(
        _moe_body,
        grid=(_NT,),
        in_specs=[
            pl.BlockSpec((_TILE, _D), lambda i: (i, 0)),
            _full_spec((1, _D)), _full_spec((1, _D)), _full_spec((_D, _E)),
            _full_spec((_E, _D, _FF)), _full_spec((_E, _FF)),
            _full_spec((_E, _FF, _D)), _full_spec((_E, _D)),
            _full_spec((_D, _FF)), _full_spec((1, _FF)),
            _full_spec((_FF, _D)), _full_spec((1, _D)),
        ],
        out_specs=pl.BlockSpec((_TILE, _D), lambda i: (i, 0)),
        out_shape=jax.ShapeDtypeStruct((_T, _D), jnp.float32),
    )(h, bp['ln2_g'].reshape(1, _D), bp['ln2_b'].reshape(1, _D), cent_t,
      bp['w1'], bp['b1'], bp['w2'], bp['b2'],
      bp['ws1'], bp['bs1'].reshape(1, _FF), bp['ws2'],
      bp['bs2'].reshape(1, _D))


def _route_body(h_ref, g2_ref, b2_ref, cent_ref, ws1_ref, bs1_ref, ws2_ref,
                bs2_ref, u2_ref, hp_ref, eid_ref, gate_ref):
    h = h_ref[...]
    u2 = _ln(h, g2_ref[...], b2_ref[...])
    logits = _f32dot(u2, cent_ref[...])
    mx = jnp.max(logits, axis=-1, keepdims=True)
    eg = jnp.exp(logits - mx)
    gsum = jnp.sum(eg, axis=-1, keepdims=True)
    iot = lax.broadcasted_iota(jnp.int32, logits.shape, 1)
    top = jnp.min(jnp.where(logits == mx, iot, _E), axis=-1, keepdims=True)
    tgate = jnp.max(jnp.where(iot == top, eg, 0.0), axis=-1,
                    keepdims=True) / gsum
    ys = _f32dot(jax.nn.gelu(_f32dot(u2, ws1_ref[...]) + bs1_ref[...]),
                 ws2_ref[...]) + bs2_ref[...]
    u2_ref[...] = u2
    hp_ref[...] = h + ys
    eid_ref[...] = top[None]
    gate_ref[...] = tgate[None]


def _moe_route(h, bp, cent_t):
    outs = (jax.ShapeDtypeStruct((_T, _D), jnp.float32),
            jax.ShapeDtypeStruct((_T, _D), jnp.float32),
            jax.ShapeDtypeStruct((_NT, _TILE, 1), jnp.int32),
            jax.ShapeDtypeStruct((_NT, _TILE, 1), jnp.float32))
    return pl.pallas_call(
        _route_body,
        grid=(_NT,),
        in_specs=[
            pl.BlockSpec((_TILE, _D), lambda i: (i, 0)),
            _full_spec((1, _D)), _full_spec((1, _D)), _full_spec((_D, _E)),
            _full_spec((_D, _FF)), _full_spec((1, _FF)),
            _full_spec((_FF, _D)), _full_spec((1, _D)),
        ],
        out_specs=[
            pl.BlockSpec((_TILE, _D), lambda i: (i, 0)),
            pl.BlockSpec((_TILE, _D), lambda i: (i, 0)),
            pl.BlockSpec((1, _TILE, 1), lambda i: (i, 0, 0)),
            pl.BlockSpec((1, _TILE, 1), lambda i: (i, 0, 0)),
        ],
        out_shape=outs,
    )(h, bp['ln2_g'].reshape(1, _D), bp['ln2_b'].reshape(1, _D), cent_t,
      bp['ws1'], bp['bs1'].reshape(1, _FF), bp['ws2'],
      bp['bs2'].reshape(1, _D))


def _sc_mesh():
    return plsc.VectorSubcoreMesh(core_axis_name="c", subcore_axis_name="s")


def _wid():
    return lax.axis_index("s") * _NC + lax.axis_index("c")


def _pos_a_body(eid_ref, pie_ref, starts_ref, run_ref):
    i = pl.program_id(0)

    @pl.when(i == 0)
    def _():
        run_ref[...] = jnp.zeros((1, 8), jnp.float32)

    e2 = eid_ref[0]
    onehot = (e2 == lax.broadcasted_iota(jnp.int32, (_TILE, 8), 1)
              ).astype(jnp.float32)
    ri = lax.broadcasted_iota(jnp.int32, (_TILE, _TILE), 0)
    ci = lax.broadcasted_iota(jnp.int32, (_TILE, _TILE), 1)
    tril = (ri > ci).astype(jnp.float32)
    rank = lax.dot_general(tril, onehot, (((1,), (0,)), ((), ())),
                           precision=lax.Precision.HIGHEST,
                           preferred_element_type=jnp.float32)
    pie_ref[...] = jnp.sum(onehot * (rank + run_ref[...]), axis=1,
                           keepdims=True)
    run_ref[...] = run_ref[...] + jnp.sum(onehot, axis=0, keepdims=True)

    @pl.when(i == _NT - 1)
    def _():
        tot = run_ref[...]
        cu = lax.broadcasted_iota(jnp.int32, (8, 8), 0)
        cv = lax.broadcasted_iota(jnp.int32, (8, 8), 1)
        sm = (cu < cv).astype(jnp.float32)
        starts_ref[...] = lax.dot_general(
            tot, sm, (((1,), (0,)), ((), ())),
            precision=lax.Precision.HIGHEST,
            preferred_element_type=jnp.float32)

def _pos_a(eid):
    return pl.pallas_call(
        _pos_a_body,
        grid=(_NT,),
        in_specs=[pl.BlockSpec((1, _TILE, 1), lambda i: (i, 0, 0))],
        out_specs=[pl.BlockSpec((_TILE, 1), lambda i: (i, 0)),
                   pl.BlockSpec((1, 8), lambda i: (0, 0))],
        out_shape=(jax.ShapeDtypeStruct((_T, 1), jnp.float32),
                   jax.ShapeDtypeStruct((1, 8), jnp.float32)),
        scratch_shapes=[pltpu.VMEM((1, 8), jnp.float32)],
    )(eid)


def _pos_b_body(eid_ref, pie_ref, starts_ref, pos_ref, sti_ref):
    e2 = eid_ref[0]
    onehot = (e2 == lax.broadcasted_iota(jnp.int32, (_TILE, 8), 1)
              ).astype(jnp.float32)
    st = jnp.sum(onehot * starts_ref[...], axis=1, keepdims=True)
    pos_ref[...] = (pie_ref[...] + st).astype(jnp.int32)

    @pl.when(pl.program_id(0) == 0)
    def _():
        sti_ref[...] = starts_ref[...].astype(jnp.int32)


def _pos_b(eid, pie, starts_f):
    return pl.pallas_call(
        _pos_b_body,
        grid=(_NT,),
        in_specs=[pl.BlockSpec((1, _TILE, 1), lambda i: (i, 0, 0)),
                  pl.BlockSpec((_TILE, 1), lambda i: (i, 0)),
                  _full_spec((1, 8))],
        out_specs=[pl.BlockSpec((_TILE, 1), lambda i: (i, 0)),
                   pl.BlockSpec((1, 8), lambda i: (0, 0))],
        out_shape=(jax.ShapeDtypeStruct((_T, 1), jnp.int32),
                   jax.ShapeDtypeStruct((1, 8), jnp.int32)),
    )(eid, pie, starts_f)


def _sc_scatter(pos, u2):
    @functools.partial(
        pl.kernel, mesh=_sc_mesh(),
        out_type=jax.ShapeDtypeStruct((_T, _D), jnp.float32),
        scratch_types=[pltpu.VMEM((_NCH, _CC), jnp.int32),
                       pltpu.VMEM((_CC, _D), jnp.float32),
                       pltpu.SemaphoreType.DMA],
    )
    def k(pos_hbm, u2_hbm, u2s_hbm, pos2, rows_v, sem):
        base = _wid() * _CHUNK
        for c in range(_NCH):
            pltpu.sync_copy(pos_hbm.at[pl.ds(base + c * _CC, _CC)],
                            pos2.at[c])
            pltpu.sync_copy(u2_hbm.at[pl.ds(base + c * _CC, _CC)], rows_v)
            pltpu.async_copy(rows_v, u2s_hbm.at[pos2.at[c]], sem).wait()

    return k(pos, u2)


def _sc_unsort(pos, ys):
    @functools.partial(
        pl.kernel, mesh=_sc_mesh(),
        out_type=jax.ShapeDtypeStruct((_T, _D), jnp.float32),
        scratch_types=[pltpu.VMEM((_NCH, _CC), jnp.int32),
                       pltpu.VMEM((_CC, _D), jnp.float32),
                       pltpu.SemaphoreType.DMA],
    )
    def k(pos_hbm, ys_hbm, yu_hbm, pos2, rows_v, sem):
        base = _wid() * _CHUNK
        for c in range(_NCH):
            pltpu.sync_copy(pos_hbm.at[pl.ds(base + c * _CC, _CC)],
                            pos2.at[c])
            pltpu.async_copy(ys_hbm.at[pos2.at[c]], rows_v, sem).wait()
            pltpu.sync_copy(rows_v, yu_hbm.at[pl.ds(base + c * _CC, _CC)])

    return k(pos, ys)


def _group_body(starts_ref, u_ref, w1_ref, b1_ref, w2_ref, b2e_ref, o_ref,
                acc_ref):
    t0 = pl.program_id(0) * _TILE
    u = u_ref[...]
    b1 = b1_ref[...]
    b2e = b2e_ref[...]
    rowid = t0 + lax.broadcasted_iota(jnp.int32, (_TILE, 1), 0)
    acc_ref[...] = jnp.zeros((_TILE, _D), jnp.float32)
    for e in range(_E):
        lo = jnp.maximum(starts_ref[e], t0)
        hi = jnp.minimum(starts_ref[e + 1], t0 + _TILE)

        @pl.when(lo < hi)
        def _(e=e, lo=lo, hi=hi):
            t1 = jax.nn.gelu(_f32dot(u, w1_ref[e]) + b1[e][None])
            t2 = _f32dot(t1, w2_ref[e]) + b2e[e][None]
            m = (rowid >= lo) & (rowid < hi)
            acc_ref[...] += jnp.where(m, t2, 0.0)

    o_ref[...] = acc_ref[...]


def _moe_group(u2s, starts, bp):
    grid_spec = pltpu.PrefetchScalarGridSpec(
        num_scalar_prefetch=1,
        grid=(_NT,),
        in_specs=[
            pl.BlockSpec((_TILE, _D), lambda i, s: (i, 0)),
            pl.BlockSpec((_E, _D, _FF), lambda i, s: (0, 0, 0)),
            pl.BlockSpec((_E, _FF), lambda i, s: (0, 0)),
            pl.BlockSpec((_E, _FF, _D), lambda i, s: (0, 0, 0)),
            pl.BlockSpec((_E, _D), lambda i, s: (0, 0)),
        ],
        out_specs=pl.BlockSpec((_TILE, _D), lambda i, s: (i, 0)),
        scratch_shapes=[pltpu.VMEM((_TILE, _D), jnp.float32)],
    )
    return pl.pallas_call(
        _group_body,
        grid_spec=grid_spec,
        out_shape=jax.ShapeDtypeStruct((_T, _D), jnp.float32),
    )(starts, u2s, bp['w1'], bp['b1'], bp['w2'], bp['b2'])


def _qkv_combine_body(hp_ref, yu_ref, gate_ref, g_ref, b_ref, w_ref, bias_ref,
                      o_ref):
    h1 = hp_ref[...] + gate_ref[...] * yu_ref[...]
    u = _ln(h1, g_ref[...], b_ref[...])
    o_ref[...] = _f32dot(u, w_ref[...]) + bias_ref[...]


def _qkv_combine(hplus, yu, gate, bp):
    return pl.pallas_call(
        _qkv_combine_body,
        grid=(_NT,),
        in_specs=[
            pl.BlockSpec((_TILE, _D), lambda i: (i, 0)),
            pl.BlockSpec((_TILE, _D), lambda i: (i, 0)),
            pl.BlockSpec((_TILE, 1), lambda i: (i, 0)),
            _full_spec((1, _D)), _full_spec((1, _D)),
            _full_spec((_D, 3 * _D)), _full_spec((1, 3 * _D)),
        ],
        out_specs=pl.BlockSpec((_TILE, 3 * _D), lambda i: (i, 0)),
        out_shape=jax.ShapeDtypeStruct((_T, 3 * _D), jnp.float32),
    )(hplus, yu, gate, bp['ln1_g'].reshape(1, _D), bp['ln1_b'].reshape(1, _D),
      bp['wqkv'], bp['bqkv'].reshape(1, 3 * _D))


def _final_body(hc_ref, yu_ref, gt_ref, a2_ref, wo_ref, bo_ref, g2_ref,
                b2_ref, cent_ref, w1_ref, b1_ref, w2_ref, b2e_ref, ws1_ref,
                bs1_ref, ws2_ref, bs2_ref, fcw_ref, fcb_ref, o_ref):
    c = (hc_ref[...] + gt_ref[...] * yu_ref[...]
         + _f32dot(a2_ref[...], wo_ref[...]) + bo_ref[...])
    o = _moe_math(c, g2_ref[...], b2_ref[...], cent_ref[...], w1_ref[...],
                  b1_ref[...], w2_ref[...], b2e_ref[...], ws1_ref[...],
                  bs1_ref[...], ws2_ref[...], bs2_ref[...])
    o_ref[...] = _f32dot(o, fcw_ref[...]) + fcb_ref[...]


def _final(hcls, yu_cls, gate_cls, att2, bp, cent_t, p):
    bb = 512
    return pl.pallas_call(
        _final_body,
        grid=(_B // bb,),
        in_specs=[
            pl.BlockSpec((bb, _D), lambda i: (i, 0)),
            pl.BlockSpec((bb, _D), lambda i: (i, 0)),
            pl.BlockSpec((bb, 1), lambda i: (i, 0)),
            pl.BlockSpec((bb, _D), lambda i: (i, 0)),
            _full_spec((_D, _D)), _full_spec((1, _D)),
            _full_spec((1, _D)), _full_spec((1, _D)), _full_spec((_D, _E)),
            _full_spec((_E, _D, _FF)), _full_spec((_E, _FF)),
            _full_spec((_E, _FF, _D)), _full_spec((_E, _D)),
            _full_spec((_D, _FF)), _full_spec((1, _FF)),
            _full_spec((_FF, _D)), _full_spec((1, _D)),
            _full_spec((_D, 3)), _full_spec((1, 3)),
        ],
        out_specs=pl.BlockSpec((bb, 3), lambda i: (i, 0)),
        out_shape=jax.ShapeDtypeStruct((_B, 3), jnp.float32),
    )(hcls, yu_cls, gate_cls, att2, bp['wo'], bp['bo'].reshape(1, _D),
      bp['ln2_g'].reshape(1, _D), bp['ln2_b'].reshape(1, _D), cent_t,
      bp['w1'], bp['b1'], bp['w2'], bp['b2'],
      bp['ws1'], bp['bs1'].reshape(1, _FF), bp['ws2'],
      bp['bs2'].reshape(1, _D), p['fc_w'], p['fc_b'].reshape(1, 3))


def kernel(x, params):
    p = params
    b0, b1 = p['blocks']
    cent_t = p['centroids'].T

    h0 = _embed(x, p)
    hf = h0.reshape(_T, _D)

    qkv1 = _qkv(hf, b0)
    att1 = _attn1(qkv1.reshape(_B, _S, 3 * _D))
    h1a = _proj(hf, att1.reshape(_T, _D), b0)

    u2, hplus, eid, gate = _moe_route(h1a, b0, cent_t)
    pie, starts_f = _pos_a(eid)
    pos2d, starts_i = _pos_b(eid, pie, starts_f)
    pos = pos2d.reshape(_T)
    starts = starts_i.reshape(8)
    u2s = _sc_scatter(pos, u2)
    ys = _moe_group(u2s, starts, b0)
    yu = _sc_unsort(pos, ys)
    gate_flat = gate.reshape(_T, 1)

    qkv2 = _qkv_combine(hplus, yu, gate_flat, b1)
    att2 = _attn2(qkv2.reshape(_B, _S, 3 * _D))

    hcls = hplus.reshape(_B, _S, _D)[:, 0, :]
    yu_cls = yu.reshape(_B, _S, _D)[:, 0, :]
    gate_cls = gate_flat.reshape(_B, _S)[:, 0:1]
    return _final(hcls, yu_cls, gate_cls, att2.reshape(_B, _D), b1, cent_t, p)


# speedup vs baseline: 1.0413x; 1.0413x over previous
import functools
import math

import jax
import jax.numpy as jnp
from jax import lax
from jax.experimental import pallas as pl
from jax.experimental.pallas import tpu as pltpu
from jax.experimental.pallas import tpu_sc as plsc

_B, _NF, _D, _FF, _E, _H = 1024, 20, 256, 512, 5, 8
_S = _NF + 1
_T = _B * _S
_DH = _D // _H
_TILE = 512
_NT = _T // _TILE
_G1 = 8
_G2 = 16
_NEG = -1e30

_NC, _NS = 2, 16
_NW = _NC * _NS
_CHUNK = _T // _NW
_NVR = _CHUNK // 16
_CC = 112
_NCH = _CHUNK // _CC
_VPC = _CC // 16


def _f32dot(a, b):
    return jnp.dot(a.astype(jnp.bfloat16), b.astype(jnp.bfloat16),
                   preferred_element_type=jnp.float32)


def _bdg(a, b, dims):
    return lax.dot_general(a.astype(jnp.bfloat16), b.astype(jnp.bfloat16),
                           dims, preferred_element_type=jnp.float32)


def _ln(x, g, b):
    m = jnp.mean(x, axis=-1, keepdims=True)
    v = jnp.mean((x - m) ** 2, axis=-1, keepdims=True)
    return (x - m) / jnp.sqrt(v + 1e-5) * g + b


def _full_spec(shape):
    n = len(shape)
    return pl.BlockSpec(shape, lambda i, _n=n: (0,) * _n)


def _div21(x):
    return (x * 3121) >> 16


def _embed_body(x_ref, w_ref, b_ref, cls_ref, g_ref, bb_ref, o_ref):
    x = x_ref[...]
    w = w_ref[...][None]
    b = b_ref[...][None]
    feat = x[:, :, None] * w + b
    cls = jnp.broadcast_to(cls_ref[...][None], (x.shape[0], 1, _D))
    h = jnp.concatenate([cls, feat], axis=1)
    o_ref[...] = _ln(h, g_ref[...][None], bb_ref[...][None])


def _embed(x, p):
    bb = 256
    return pl.pallas_call(
        _embed_body,
        grid=(_B // bb,),
        in_specs=[
            pl.BlockSpec((bb, _NF), lambda i: (i, 0)),
            _full_spec((1, _D)), _full_spec((1, _D)), _full_spec((1, _D)),
            _full_spec((1, _D)), _full_spec((1, _D)),
        ],
        out_specs=pl.BlockSpec((bb, _S, _D), lambda i: (i, 0, 0)),
        out_shape=jax.ShapeDtypeStruct((_B, _S, _D), jnp.float32),
    )(x, p['con_w'].reshape(1, _D), p['con_b'].reshape(1, _D),
      p['cls'].reshape(1, _D), p['norm_g'].reshape(1, _D),
      p['norm_b'].reshape(1, _D))


def _qkv_body(h_ref, g_ref, b_ref, w_ref, bias_ref, o_ref):
    u = _ln(h_ref[...], g_ref[...], b_ref[...])
    o_ref[...] = _f32dot(u, w_ref[...]) + bias_ref[...]


def _qkv(h, bp):
    return pl.pallas_call(
        _qkv_body,
        grid=(_NT,),
        in_specs=[
            pl.BlockSpec((_TILE, _D), lambda i: (i, 0)),
            _full_spec((1, _D)), _full_spec((1, _D)),
            _full_spec((_D, 3 * _D)), _full_spec((1, 3 * _D)),
        ],
        out_specs=pl.BlockSpec((_TILE, 3 * _D), lambda i: (i, 0)),
        out_shape=jax.ShapeDtypeStruct((_T, 3 * _D), jnp.float32),
    )(h, bp['ln1_g'].reshape(1, _D), bp['ln1_b'].reshape(1, _D),
      bp['wqkv'], bp['bqkv'].reshape(1, 3 * _D))


def _split_heads(t):
    return jnp.concatenate([t[:, :, _DH * h:_DH * (h + 1)] for h in range(_H)],
                           axis=1)


def _attn1_body(qkv_ref, o_ref):
    u = qkv_ref[...]
    qh = _split_heads(u[:, :, :_D])
    kh = _split_heads(u[:, :, _D:2 * _D])
    vh = _split_heads(u[:, :, 2 * _D:])
    s = _bdg(qh, kh, (((2,), (2,)), ((0,), (0,))))
    hs = _H * _S
    rh = _div21(lax.broadcasted_iota(jnp.int32, (hs, hs), 0))
    ch = _div21(lax.broadcasted_iota(jnp.int32, (hs, hs), 1))
    s = jnp.where((rh == ch)[None], s * (1.0 / math.sqrt(_DH)), _NEG)
    s = s - jnp.max(s, axis=-1, keepdims=True)
    e = jnp.exp(s)
    a = e / jnp.sum(e, axis=-1, keepdims=True)
    o = _bdg(a, vh, (((2,), (1,)), ((0,), (0,))))
    o_ref[...] = jnp.concatenate(
        [o[:, _S * h:_S * (h + 1), :] for h in range(_H)], axis=2)


def _attn1(qkv):
    return pl.pallas_call(
        _attn1_body,
        grid=(_B // _G1,),
        in_specs=[pl.BlockSpec((_G1, _S, 3 * _D), lambda i: (i, 0, 0))],
        out_specs=pl.BlockSpec((_G1, _S, _D), lambda i: (i, 0, 0)),
        out_shape=jax.ShapeDtypeStruct((_B, _S, _D), jnp.float32),
    )(qkv)


def _attn2_body(qkv_ref, o_ref):
    u = qkv_ref[...]
    q0 = u[:, 0:1, :_D]
    qh = _split_heads(q0)
    kh = _split_heads(u[:, :, _D:2 * _D])
    vh = _split_heads(u[:, :, 2 * _D:])
    s = _bdg(qh, kh, (((2,), (2,)), ((0,), (0,))))
    hs = _H * _S
    rh = lax.broadcasted_iota(jnp.int32, (_H, hs), 0)
    ch = _div21(lax.broadcasted_iota(jnp.int32, (_H, hs), 1))
    s = jnp.where((rh == ch)[None], s * (1.0 / math.sqrt(_DH)), _NEG)
    s = s - jnp.max(s, axis=-1, keepdims=True)
    e = jnp.exp(s)
    a = e / jnp.sum(e, axis=-1, keepdims=True)
    o = _bdg(a, vh, (((2,), (1,)), ((0,), (0,))))
    o_ref[...] = jnp.concatenate(
        [o[:, h:h + 1, :] for h in range(_H)], axis=2)


def _attn2(qkv):
    return pl.pallas_call(
        _attn2_body,
        grid=(_B // _G2,),
        in_specs=[pl.BlockSpec((_G2, _S, 3 * _D), lambda i: (i, 0, 0))],
        out_specs=pl.BlockSpec((_G2, 1, _D), lambda i: (i, 0, 0)),
        out_shape=jax.ShapeDtypeStruct((_B, 1, _D), jnp.float32),
    )(qkv)


def _proj_body(h_ref, a_ref, w_ref, b_ref, o_ref):
    o_ref[...] = h_ref[...] + _f32dot(a_ref[...], w_ref[...]) + b_ref[...]


def _proj(h, att, bp):
    return pl.pallas_call(
        _proj_body,
        grid=(_NT,),
        in_specs=[
            pl.BlockSpec((_TILE, _D), lambda i: (i, 0)),
            pl.BlockSpec((_TILE, _D), lambda i: (i, 0)),
            _full_spec((_D, _D)), _full_spec((1, _D)),
        ],
        out_specs=pl.BlockSpec((_TILE, _D), lambda i: (i, 0)),
        out_shape=jax.ShapeDtypeStruct((_T, _D), jnp.float32),
    )(h, att, bp['wo'], bp['bo'].reshape(1, _D))


def _moe_math(h, g2, b2, cent, w1, b1, w2, b2e, ws1, bs1, ws2, bs2):
    u2 = _ln(h, g2, b2)
    logits = _f32dot(u2, cent)
    mx = jnp.max(logits, axis=-1, keepdims=True)
    eg = jnp.exp(logits - mx)
    gate = eg / jnp.sum(eg, axis=-1, keepdims=True)
    iot = lax.broadcasted_iota(jnp.int32, logits.shape, 1)
    top = jnp.min(jnp.where(logits == mx, iot, _E), axis=-1, keepdims=True)
    acc = jnp.zeros_like(h)
    for e in range(_E):
        t1 = jax.nn.gelu(_f32dot(u2, w1[e]) + b1[e][None])
        t2 = _f32dot(t1, w2[e]) + b2e[e][None]
        sel = jnp.where(top == e, gate[:, e:e + 1], 0.0)
        acc = acc + sel * t2
    ys = _f32dot(jax.nn.gelu(_f32dot(u2, ws1) + bs1), ws2) + bs2
    return h + acc + ys


def _moe_body(h_ref, g2_ref, b2_ref, cent_ref, w1_ref, b1_ref, w2_ref,
              b2e_ref, ws1_ref, bs1_ref, ws2_ref, bs2_ref, o_ref):
    o_ref[...] = _moe_math(
        h_ref[...], g2_ref[...], b2_ref[...], cent_ref[...], w1_ref[...],
        b1_ref[...], w2_ref[...], b2e_ref[...], ws1_ref[...], bs1_ref[...],
        ws2_ref[...], bs2_ref[...])


def _moe_dense(h, bp, cent_t):
    return pl.pallas_call(
        _moe_body,
        grid=(_NT,),
        in_specs=[
            pl.BlockSpec((_TILE, _D), lambda i: (i, 0)),
            _full_spec((1, _D)), _full_spec((1, _D)), _full_spec((_D, _E)),
            _full_spec((_E, _D, _FF)), _full_spec((_E, _FF)),
            _full_spec((_E, _FF, _D)), _full_spec((_E, _D)),
            _full_spec((_D, _FF)), _full_spec((1, _FF)),
            _full_spec((_FF, _D)), _full_spec((1, _D)),
        ],
        out_specs=pl.BlockSpec((_TILE, _D), lambda i: (i, 0)),
        out_shape=jax.ShapeDtypeStruct((_T, _D), jnp.float32),
    )(h, bp['ln2_g'].reshape(1, _D), bp['ln2_b'].reshape(1, _D), cent_t,
      bp['w1'], bp['b1'], bp['w2'], bp['b2'],
      bp['ws1'], bp['bs1'].reshape(1, _FF), bp['ws2'],
      bp['bs2'].reshape(1, _D))


def _route_body(hf_ref, a_ref, wo_ref, bo_ref, g2_ref, b2_ref, cent_ref,
                ws1_ref, bs1_ref, ws2_ref, bs2_ref, u2_ref, hp_ref, eid_ref,
                gate_ref):
    h = hf_ref[...] + _f32dot(a_ref[...], wo_ref[...]) + bo_ref[...]
    u2 = _ln(h, g2_ref[...], b2_ref[...])
    logits = _f32dot(u2, cent_ref[...])
    mx = jnp.max(logits, axis=-1, keepdims=True)
    eg = jnp.exp(logits - mx)
    gsum = jnp.sum(eg, axis=-1, keepdims=True)
    iot = lax.broadcasted_iota(jnp.int32, logits.shape, 1)
    top = jnp.min(jnp.where(logits == mx, iot, _E), axis=-1, keepdims=True)
    tgate = jnp.max(jnp.where(iot == top, eg, 0.0), axis=-1,
                    keepdims=True) / gsum
    ys = _f32dot(jax.nn.gelu(_f32dot(u2, ws1_ref[...]) + bs1_ref[...]),
                 ws2_ref[...]) + bs2_ref[...]
    u2_ref[...] = u2
    hp_ref[...] = h + ys
    eid_ref[...] = top[None]
    gate_ref[...] = tgate[None]


def _moe_route(hf, att, bp, cent_t):
    outs = (jax.ShapeDtypeStruct((_T, _D), jnp.float32),
            jax.ShapeDtypeStruct((_T, _D), jnp.float32),
            jax.ShapeDtypeStruct((_NT, _TILE, 1), jnp.int32),
            jax.ShapeDtypeStruct((_NT, _TILE, 1), jnp.float32))
    return pl.pallas_call(
        _route_body,
        grid=(_NT,),
        in_specs=[
            pl.BlockSpec((_TILE, _D), lambda i: (i, 0)),
            pl.BlockSpec((_TILE, _D), lambda i: (i, 0)),
            _full_spec((_D, _D)), _full_spec((1, _D)),
            _full_spec((1, _D)), _full_spec((1, _D)), _full_spec((_D, _E)),
            _full_spec((_D, _FF)), _full_spec((1, _FF)),
            _full_spec((_FF, _D)), _full_spec((1, _D)),
        ],
        out_specs=[
            pl.BlockSpec((_TILE, _D), lambda i: (i, 0)),
            pl.BlockSpec((_TILE, _D), lambda i: (i, 0)),
            pl.BlockSpec((1, _TILE, 1), lambda i: (i, 0, 0)),
            pl.BlockSpec((1, _TILE, 1), lambda i: (i, 0, 0)),
        ],
        out_shape=outs,
    )(hf, att, bp['wo'], bp['bo'].reshape(1, _D),
      bp['ln2_g'].reshape(1, _D), bp['ln2_b'].reshape(1, _D), cent_t,
      bp['ws1'], bp['bs1'].reshape(1, _FF), bp['ws2'],
      bp['bs2'].reshape(1, _D))


def _sc_mesh():
    return plsc.VectorSubcoreMesh(core_axis_name="c", subcore_axis_name="s")


def _wid():
    return lax.axis_index("s") * _NC + lax.axis_index("c")


def _pos_a_body(eid_ref, pie_ref, starts_ref, run_ref):
    i = pl.program_id(0)

    @pl.when(i == 0)
    def _():
        run_ref[...] = jnp.zeros((1, 8), jnp.float32)

    e2 = eid_ref[0]
    onehot = (e2 == lax.broadcasted_iota(jnp.int32, (_TILE, 8), 1)
              ).astype(jnp.float32)
    ri = lax.broadcasted_iota(jnp.int32, (_TILE, _TILE), 0)
    ci = lax.broadcasted_iota(jnp.int32, (_TILE, _TILE), 1)
    tril = (ri > ci).astype(jnp.float32)
    rank = lax.dot_general(tril, onehot, (((1,), (0,)), ((), ())),
                           precision=lax.Precision.HIGHEST,
                           preferred_element_type=jnp.float32)
    pie_ref[...] = jnp.sum(onehot * (rank + run_ref[...]), axis=1,
                           keepdims=True)
    run_ref[...] = run_ref[...] + jnp.sum(onehot, axis=0, keepdims=True)

    @pl.when(i == _NT - 1)
    def _():
        tot = run_ref[...]
        cu = lax.broadcasted_iota(jnp.int32, (8, 8), 0)
        cv = lax.broadcasted_iota(jnp.int32, (8, 8), 1)
        sm = (cu < cv).astype(jnp.float32)
        starts_ref[...] = lax.dot_general(
            tot, sm, (((1,), (0,)), ((), ())),
            precision=lax.Precision.HIGHEST,
            preferred_element_type=jnp.float32)

def _pos_a(eid):
    return pl.pallas_call(
        _pos_a_body,
        grid=(_NT,),
        in_specs=[pl.BlockSpec((1, _TILE, 1), lambda i: (i, 0, 0))],
        out_specs=[pl.BlockSpec((_TILE, 1), lambda i: (i, 0)),
                   pl.BlockSpec((1, 8), lambda i: (0, 0))],
        out_shape=(jax.ShapeDtypeStruct((_T, 1), jnp.float32),
                   jax.ShapeDtypeStruct((1, 8), jnp.float32)),
        scratch_shapes=[pltpu.VMEM((1, 8), jnp.float32)],
    )(eid)


def _pos_b_body(eid_ref, pie_ref, starts_ref, pos_ref, sti_ref):
    e2 = eid_ref[0]
    onehot = (e2 == lax.broadcasted_iota(jnp.int32, (_TILE, 8), 1)
              ).astype(jnp.float32)
    st = jnp.sum(onehot * starts_ref[...], axis=1, keepdims=True)
    pos_ref[...] = (pie_ref[...] + st).astype(jnp.int32)

    @pl.when(pl.program_id(0) == 0)
    def _():
        sti_ref[...] = starts_ref[...].astype(jnp.int32)


def _pos_b(eid, pie, starts_f):
    return pl.pallas_call(
        _pos_b_body,
        grid=(_NT,),
        in_specs=[pl.BlockSpec((1, _TILE, 1), lambda i: (i, 0, 0)),
                  pl.BlockSpec((_TILE, 1), lambda i: (i, 0)),
                  _full_spec((1, 8))],
        out_specs=[pl.BlockSpec((_TILE, 1), lambda i: (i, 0)),
                   pl.BlockSpec((1, 8), lambda i: (0, 0))],
        out_shape=(jax.ShapeDtypeStruct((_T, 1), jnp.int32),
                   jax.ShapeDtypeStruct((1, 8), jnp.int32)),
    )(eid, pie, starts_f)


def _sc_scatter(pos3, u2):
    @functools.partial(
        pl.kernel, mesh=_sc_mesh(),
        out_type=jax.ShapeDtypeStruct((_T, _D), jnp.float32),
        scratch_types=[pltpu.VMEM((_NCH, _CC), jnp.int32)]
        + [pltpu.VMEM((_CC, _D), jnp.float32)] * 3
        + [pltpu.SemaphoreType.DMA] * 6,
    )
    def k(pos_hbm, u2_hbm, u2s_hbm, pos_v, r0, r1, r2,
          l0, l1, l2, s0, s1, s2):
        wid = _wid()
        base = wid * _CHUNK
        pltpu.sync_copy(pos_hbm.at[wid], pos_v)
        rows = (r0, r1, r2)
        lsem = (l0, l1, l2)
        ssem = (s0, s1, s2)
        ld, st = {}, {}
        for c in range(min(2, _NCH)):
            ld[c] = pltpu.async_copy(
                u2_hbm.at[pl.ds(base + c * _CC, _CC)], rows[c % 3],
                lsem[c % 3])
        for c in range(_NCH):
            if c + 2 < _NCH:
                if c - 1 >= 0:
                    st[c - 1].wait()
                ld[c + 2] = pltpu.async_copy(
                    u2_hbm.at[pl.ds(base + (c + 2) * _CC, _CC)],
                    rows[(c + 2) % 3], lsem[(c + 2) % 3])
            ld[c].wait()
            st[c] = pltpu.async_copy(rows[c % 3], u2s_hbm.at[pos_v.at[c]],
                                     ssem[c % 3])
        for c in range(max(0, _NCH - 3), _NCH):
            st[c].wait()

    return k(pos3, u2)


def _sc_unsort(pos3, ys):
    @functools.partial(
        pl.kernel, mesh=_sc_mesh(),
        out_type=jax.ShapeDtypeStruct((_T, _D), jnp.float32),
        scratch_types=[pltpu.VMEM((_NCH, _CC), jnp.int32)]
        + [pltpu.VMEM((_CC, _D), jnp.float32)] * 3
        + [pltpu.SemaphoreType.DMA] * 6,
    )
    def k(pos_hbm, ys_hbm, yu_hbm, pos_v, r0, r1, r2,
          g0, g1, g2, s0, s1, s2):
        wid = _wid()
        base = wid * _CHUNK
        pltpu.sync_copy(pos_hbm.at[wid], pos_v)
        rows = (r0, r1, r2)
        gsem = (g0, g1, g2)
        ssem = (s0, s1, s2)
        gd, st = {}, {}
        for c in range(min(2, _NCH)):
            gd[c] = pltpu.async_copy(ys_hbm.at[pos_v.at[c]], rows[c % 3],
                                     gsem[c % 3])
        for c in range(_NCH):
            if c + 2 < _NCH:
                if c - 1 >= 0:
                    st[c - 1].wait()
                gd[c + 2] = pltpu.async_copy(
                    ys_hbm.at[pos_v.at[c + 2]], rows[(c + 2) % 3],
                    gsem[(c + 2) % 3])
            gd[c].wait()
            st[c] = pltpu.async_copy(
                rows[c % 3], yu_hbm.at[pl.ds(base + c * _CC, _CC)],
                ssem[c % 3])
        for c in range(max(0, _NCH - 3), _NCH):
            st[c].wait()

    return k(pos3, ys)


def _group_body(starts_ref, u_ref, w1_ref, b1_ref, w2_ref, b2e_ref, o_ref,
                acc_ref):
    t0 = pl.program_id(0) * _TILE
    u = u_ref[...]
    b1 = b1_ref[...]
    b2e = b2e_ref[...]
    rowid = t0 + lax.broadcasted_iota(jnp.int32, (_TILE, 1), 0)
    acc_ref[...] = jnp.zeros((_TILE, _D), jnp.float32)
    for e in range(_E):
        lo = jnp.maximum(starts_ref[e], t0)
        hi = jnp.minimum(starts_ref[e + 1], t0 + _TILE)

        @pl.when(lo < hi)
        def _(e=e, lo=lo, hi=hi):
            t1 = jax.nn.gelu(_f32dot(u, w1_ref[e]) + b1[e][None])
            t2 = _f32dot(t1, w2_ref[e]) + b2e[e][None]
            m = (rowid >= lo) & (rowid < hi)
            acc_ref[...] += jnp.where(m, t2, 0.0)

    o_ref[...] = acc_ref[...]


def _moe_group(u2s, starts, bp):
    grid_spec = pltpu.PrefetchScalarGridSpec(
        num_scalar_prefetch=1,
        grid=(_NT,),
        in_specs=[
            pl.BlockSpec((_TILE, _D), lambda i, s: (i, 0)),
            pl.BlockSpec((_E, _D, _FF), lambda i, s: (0, 0, 0)),
            pl.BlockSpec((_E, _FF), lambda i, s: (0, 0)),
            pl.BlockSpec((_E, _FF, _D), lambda i, s: (0, 0, 0)),
            pl.BlockSpec((_E, _D), lambda i, s: (0, 0)),
        ],
        out_specs=pl.BlockSpec((_TILE, _D), lambda i, s: (i, 0)),
        scratch_shapes=[pltpu.VMEM((_TILE, _D), jnp.float32)],
    )
    return pl.pallas_call(
        _group_body,
        grid_spec=grid_spec,
        out_shape=jax.ShapeDtypeStruct((_T, _D), jnp.float32),
    )(starts, u2s, bp['w1'], bp['b1'], bp['w2'], bp['b2'])


def _qkv_combine_body(hp_ref, yu_ref, gate_ref, g_ref, b_ref, w_ref, bias_ref,
                      o_ref):
    h1 = hp_ref[...] + gate_ref[...] * yu_ref[...]
    u = _ln(h1, g_ref[...], b_ref[...])
    o_ref[...] = _f32dot(u, w_ref[...]) + bias_ref[...]


def _qkv_combine(hplus, yu, gate, bp):
    return pl.pallas_call(
        _qkv_combine_body,
        grid=(_NT,),
        in_specs=[
            pl.BlockSpec((_TILE, _D), lambda i: (i, 0)),
            pl.BlockSpec((_TILE, _D), lambda i: (i, 0)),
            pl.BlockSpec((_TILE, 1), lambda i: (i, 0)),
            _full_spec((1, _D)), _full_spec((1, _D)),
            _full_spec((_D, 3 * _D)), _full_spec((1, 3 * _D)),
        ],
        out_specs=pl.BlockSpec((_TILE, 3 * _D), lambda i: (i, 0)),
        out_shape=jax.ShapeDtypeStruct((_T, 3 * _D), jnp.float32),
    )(hplus, yu, gate, bp['ln1_g'].reshape(1, _D), bp['ln1_b'].reshape(1, _D),
      bp['wqkv'], bp['bqkv'].reshape(1, 3 * _D))


def _final_body(hc_ref, yu_ref, gt_ref, a2_ref, wo_ref, bo_ref, g2_ref,
                b2_ref, cent_ref, w1_ref, b1_ref, w2_ref, b2e_ref, ws1_ref,
                bs1_ref, ws2_ref, bs2_ref, fcw_ref, fcb_ref, o_ref):
    c = (hc_ref[...] + gt_ref[...] * yu_ref[...]
         + _f32dot(a2_ref[...], wo_ref[...]) + bo_ref[...])
    o = _moe_math(c, g2_ref[...], b2_ref[...], cent_ref[...], w1_ref[...],
                  b1_ref[...], w2_ref[...], b2e_ref[...], ws1_ref[...],
                  bs1_ref[...], ws2_ref[...], bs2_ref[...])
    o_ref[...] = _f32dot(o, fcw_ref[...]) + fcb_ref[...]


def _final(hcls, yu_cls, gate_cls, att2, bp, cent_t, p):
    bb = 512
    return pl.pallas_call(
        _final_body,
        grid=(_B // bb,),
        in_specs=[
            pl.BlockSpec((bb, _D), lambda i: (i, 0)),
            pl.BlockSpec((bb, _D), lambda i: (i, 0)),
            pl.BlockSpec((bb, 1), lambda i: (i, 0)),
            pl.BlockSpec((bb, _D), lambda i: (i, 0)),
            _full_spec((_D, _D)), _full_spec((1, _D)),
            _full_spec((1, _D)), _full_spec((1, _D)), _full_spec((_D, _E)),
            _full_spec((_E, _D, _FF)), _full_spec((_E, _FF)),
            _full_spec((_E, _FF, _D)), _full_spec((_E, _D)),
            _full_spec((_D, _FF)), _full_spec((1, _FF)),
            _full_spec((_FF, _D)), _full_spec((1, _D)),
            _full_spec((_D, 3)), _full_spec((1, 3)),
        ],
        out_specs=pl.BlockSpec((bb, 3), lambda i: (i, 0)),
        out_shape=jax.ShapeDtypeStruct((_B, 3), jnp.float32),
    )(hcls, yu_cls, gate_cls, att2, bp['wo'], bp['bo'].reshape(1, _D),
      bp['ln2_g'].reshape(1, _D), bp['ln2_b'].reshape(1, _D), cent_t,
      bp['w1'], bp['b1'], bp['w2'], bp['b2'],
      bp['ws1'], bp['bs1'].reshape(1, _FF), bp['ws2'],
      bp['bs2'].reshape(1, _D), p['fc_w'], p['fc_b'].reshape(1, 3))


def kernel(x, params):
    p = params
    b0, b1 = p['blocks']
    cent_t = p['centroids'].T

    h0 = _embed(x, p)
    hf = h0.reshape(_T, _D)

    qkv1 = _qkv(hf, b0)
    att1 = _attn1(qkv1.reshape(_B, _S, 3 * _D))

    u2, hplus, eid, gate = _moe_route(hf, att1.reshape(_T, _D), b0, cent_t)
    pie, starts_f = _pos_a(eid)
    pos2d, starts_i = _pos_b(eid, pie, starts_f)
    pos3 = pos2d.reshape(_NW, _NCH, _CC)
    starts = starts_i.reshape(8)
    u2s = _sc_scatter(pos3, u2)
    ys = _moe_group(u2s, starts, b0)
    yu = _sc_unsort(pos3, ys)
    gate_flat = gate.reshape(_T, 1)

    qkv2 = _qkv_combine(hplus, yu, gate_flat, b1)
    att2 = _attn2(qkv2.reshape(_B, _S, 3 * _D))

    hcls = hplus.reshape(_B, _S, _D)[:, 0, :]
    yu_cls = yu.reshape(_B, _S, _D)[:, 0, :]
    gate_cls = gate_flat.reshape(_B, _S)[:, 0:1]
    return _final(hcls, yu_cls, gate_cls, att2.reshape(_B, _D), b1, cent_t, p)


# speedup vs baseline: 1.2369x; 1.1879x over previous
import functools
import math

import jax
import jax.numpy as jnp
from jax import lax
from jax.experimental import pallas as pl
from jax.experimental.pallas import tpu as pltpu
from jax.experimental.pallas import tpu_sc as plsc

_B, _NF, _D, _FF, _E, _H = 1024, 20, 256, 512, 5, 8
_S = _NF + 1
_T = _B * _S
_DH = _D // _H
_TILE = 512
_NT = _T // _TILE
_G1 = 8
_G2 = 16
_NEG = -1e30

_NC, _NS = 2, 16
_NW = _NC * _NS
_CHUNK = _T // _NW
_NVR = _CHUNK // 16
_CC = 112
_NCH = _CHUNK // _CC
_VPC = _CC // 16


def _f32dot(a, b):
    return jnp.dot(a.astype(jnp.bfloat16), b.astype(jnp.bfloat16),
                   preferred_element_type=jnp.float32)


def _bdg(a, b, dims):
    return lax.dot_general(a.astype(jnp.bfloat16), b.astype(jnp.bfloat16),
                           dims, preferred_element_type=jnp.float32)


def _ln(x, g, b):
    m = jnp.mean(x, axis=-1, keepdims=True)
    v = jnp.mean((x - m) ** 2, axis=-1, keepdims=True)
    return (x - m) / jnp.sqrt(v + 1e-5) * g + b


def _full_spec(shape):
    n = len(shape)
    return pl.BlockSpec(shape, lambda i, _n=n: (0,) * _n)


def _div21(x):
    return (x * 3121) >> 16


_EB = 256


def _embed_body(x_ref, w_ref, b_ref, cls_ref, g_ref, bb_ref, g1_ref, b1_ref,
                wqkv_ref, bqkv_ref, h_ref, qkv_ref):
    x = x_ref[...]
    w = w_ref[...][None]
    b = b_ref[...][None]
    feat = x[:, :, None] * w + b
    cls = jnp.broadcast_to(cls_ref[...][None], (x.shape[0], 1, _D))
    h = jnp.concatenate([cls, feat], axis=1)
    h0 = _ln(h, g_ref[...][None], bb_ref[...][None])
    h_ref[...] = h0
    u = _ln(h0.reshape(_EB * _S, _D), g1_ref[...], b1_ref[...])
    qkv_ref[...] = _f32dot(u, wqkv_ref[...]) + bqkv_ref[...]


def _embed_qkv(x, p, bp):
    return pl.pallas_call(
        _embed_body,
        grid=(_B // _EB,),
        in_specs=[
            pl.BlockSpec((_EB, _NF), lambda i: (i, 0)),
            _full_spec((1, _D)), _full_spec((1, _D)), _full_spec((1, _D)),
            _full_spec((1, _D)), _full_spec((1, _D)),
            _full_spec((1, _D)), _full_spec((1, _D)),
            _full_spec((_D, 3 * _D)), _full_spec((1, 3 * _D)),
        ],
        out_specs=[
            pl.BlockSpec((_EB, _S, _D), lambda i: (i, 0, 0)),
            pl.BlockSpec((_EB * _S, 3 * _D), lambda i: (i, 0)),
        ],
        out_shape=(jax.ShapeDtypeStruct((_B, _S, _D), jnp.float32),
                   jax.ShapeDtypeStruct((_T, 3 * _D), jnp.float32)),
    )(x, p['con_w'].reshape(1, _D), p['con_b'].reshape(1, _D),
      p['cls'].reshape(1, _D), p['norm_g'].reshape(1, _D),
      p['norm_b'].reshape(1, _D),
      bp['ln1_g'].reshape(1, _D), bp['ln1_b'].reshape(1, _D),
      bp['wqkv'], bp['bqkv'].reshape(1, 3 * _D))


def _split_heads(t):
    return jnp.concatenate([t[:, :, _DH * h:_DH * (h + 1)] for h in range(_H)],
                           axis=1)


def _attn1_body(qkv_ref, o_ref):
    u = qkv_ref[...]
    qh = _split_heads(u[:, :, :_D])
    kh = _split_heads(u[:, :, _D:2 * _D])
    vh = _split_heads(u[:, :, 2 * _D:])
    s = _bdg(qh, kh, (((2,), (2,)), ((0,), (0,))))
    hs = _H * _S
    rh = _div21(lax.broadcasted_iota(jnp.int32, (hs, hs), 0))
    ch = _div21(lax.broadcasted_iota(jnp.int32, (hs, hs), 1))
    s = jnp.where((rh == ch)[None], s * (1.0 / math.sqrt(_DH)), _NEG)
    s = s - jnp.max(s, axis=-1, keepdims=True)
    e = jnp.exp(s)
    a = e / jnp.sum(e, axis=-1, keepdims=True)
    o = _bdg(a, vh, (((2,), (1,)), ((0,), (0,))))
    o_ref[...] = jnp.concatenate(
        [o[:, _S * h:_S * (h + 1), :] for h in range(_H)], axis=2)


def _attn1(qkv):
    return pl.pallas_call(
        _attn1_body,
        grid=(_B // _G1,),
        in_specs=[pl.BlockSpec((_G1, _S, 3 * _D), lambda i: (i, 0, 0))],
        out_specs=pl.BlockSpec((_G1, _S, _D), lambda i: (i, 0, 0)),
        out_shape=jax.ShapeDtypeStruct((_B, _S, _D), jnp.float32),
    )(qkv)


_CB = 64


def _combine_attn2_body(hp_ref, yu_ref, gate_ref, g_ref, b_ref, w_ref,
                        bias_ref, a2_ref, hc_ref):
    h1 = hp_ref[...] + gate_ref[...] * yu_ref[...]
    u = _ln(h1, g_ref[...], b_ref[...])
    qkv = _f32dot(u, w_ref[...]) + bias_ref[...]
    u3 = qkv.reshape(_CB, _S, 3 * _D)
    q0 = u3[:, 0:1, :_D]
    qh = _split_heads(q0)
    kh = _split_heads(u3[:, :, _D:2 * _D])
    vh = _split_heads(u3[:, :, 2 * _D:])
    s = _bdg(qh, kh, (((2,), (2,)), ((0,), (0,))))
    hs = _H * _S
    rh = lax.broadcasted_iota(jnp.int32, (_H, hs), 0)
    ch = _div21(lax.broadcasted_iota(jnp.int32, (_H, hs), 1))
    s = jnp.where((rh == ch)[None], s * (1.0 / math.sqrt(_DH)), _NEG)
    s = s - jnp.max(s, axis=-1, keepdims=True)
    e = jnp.exp(s)
    a = e / jnp.sum(e, axis=-1, keepdims=True)
    o = _bdg(a, vh, (((2,), (1,)), ((0,), (0,))))
    a2_ref[...] = jnp.concatenate(
        [o[:, h:h + 1, :] for h in range(_H)], axis=2).reshape(_CB, _D)
    hc_ref[...] = h1.reshape(_CB, _S, _D)[:, 0, :]


def _combine_attn2(hplus, yu, gate, bp):
    rows = _CB * _S
    return pl.pallas_call(
        _combine_attn2_body,
        grid=(_B // _CB,),
        in_specs=[
            pl.BlockSpec((rows, _D), lambda i: (i, 0)),
            pl.BlockSpec((rows, _D), lambda i: (i, 0)),
            pl.BlockSpec((rows, 1), lambda i: (i, 0)),
            _full_spec((1, _D)), _full_spec((1, _D)),
            _full_spec((_D, 3 * _D)), _full_spec((1, 3 * _D)),
        ],
        out_specs=[
            pl.BlockSpec((_CB, _D), lambda i: (i, 0)),
            pl.BlockSpec((_CB, _D), lambda i: (i, 0)),
        ],
        out_shape=(jax.ShapeDtypeStruct((_B, _D), jnp.float32),
                   jax.ShapeDtypeStruct((_B, _D), jnp.float32)),
    )(hplus, yu, gate, bp['ln1_g'].reshape(1, _D), bp['ln1_b'].reshape(1, _D),
      bp['wqkv'], bp['bqkv'].reshape(1, 3 * _D))


def _proj_body(h_ref, a_ref, w_ref, b_ref, o_ref):
    o_ref[...] = h_ref[...] + _f32dot(a_ref[...], w_ref[...]) + b_ref[...]


def _proj(h, att, bp):
    return pl.pallas_call(
        _proj_body,
        grid=(_NT,),
        in_specs=[
            pl.BlockSpec((_TILE, _D), lambda i: (i, 0)),
            pl.BlockSpec((_TILE, _D), lambda i: (i, 0)),
            _full_spec((_D, _D)), _full_spec((1, _D)),
        ],
        out_specs=pl.BlockSpec((_TILE, _D), lambda i: (i, 0)),
        out_shape=jax.ShapeDtypeStruct((_T, _D), jnp.float32),
    )(h, att, bp['wo'], bp['bo'].reshape(1, _D))


def _moe_math(h, g2, b2, cent, w1, b1, w2, b2e, ws1, bs1, ws2, bs2):
    u2 = _ln(h, g2, b2)
    logits = _f32dot(u2, cent)
    mx = jnp.max(logits, axis=-1, keepdims=True)
    eg = jnp.exp(logits - mx)
    gate = eg / jnp.sum(eg, axis=-1, keepdims=True)
    iot = lax.broadcasted_iota(jnp.int32, logits.shape, 1)
    top = jnp.min(jnp.where(logits == mx, iot, _E), axis=-1, keepdims=True)
    acc = jnp.zeros_like(h)
    for e in range(_E):
        t1 = jax.nn.gelu(_f32dot(u2, w1[e]) + b1[e][None])
        t2 = _f32dot(t1, w2[e]) + b2e[e][None]
        sel = jnp.where(top == e, gate[:, e:e + 1], 0.0)
        acc = acc + sel * t2
    ys = _f32dot(jax.nn.gelu(_f32dot(u2, ws1) + bs1), ws2) + bs2
    return h + acc + ys


def _moe_body(h_ref, g2_ref, b2_ref, cent_ref, w1_ref, b1_ref, w2_ref,
              b2e_ref, ws1_ref, bs1_ref, ws2_ref, bs2_ref, o_ref):
    o_ref[...] = _moe_math(
        h_ref[...], g2_ref[...], b2_ref[...], cent_ref[...], w1_ref[...],
        b1_ref[...], w2_ref[...], b2e_ref[...], ws1_ref[...], bs1_ref[...],
        ws2_ref[...], bs2_ref[...])


def _moe_dense(h, bp, cent_t):
    return pl.pallas_call(
        _moe_body,
        grid=(_NT,),
        in_specs=[
            pl.BlockSpec((_TILE, _D), lambda i: (i, 0)),
            _full_spec((1, _D)), _full_spec((1, _D)), _full_spec((_D, _E)),
            _full_spec((_E, _D, _FF)), _full_spec((_E, _FF)),
            _full_spec((_E, _FF, _D)), _full_spec((_E, _D)),
            _full_spec((_D, _FF)), _full_spec((1, _FF)),
            _full_spec((_FF, _D)), _full_spec((1, _D)),
        ],
        out_specs=pl.BlockSpec((_TILE, _D), lambda i: (i, 0)),
        out_shape=jax.ShapeDtypeStruct((_T, _D), jnp.float32),
    )(h, bp['ln2_g'].reshape(1, _D), bp['ln2_b'].reshape(1, _D), cent_t,
      bp['w1'], bp['b1'], bp['w2'], bp['b2'],
      bp['ws1'], bp['bs1'].reshape(1, _FF), bp['ws2'],
      bp['bs2'].reshape(1, _D))


def _route_body(hf_ref, a_ref, wo_ref, bo_ref, g2_ref, b2_ref, cent_ref,
                ws1_ref, bs1_ref, ws2_ref, bs2_ref, u2_ref, hp_ref, eid_ref,
                gate_ref):
    h = hf_ref[...] + _f32dot(a_ref[...], wo_ref[...]) + bo_ref[...]
    u2 = _ln(h, g2_ref[...], b2_ref[...])
    logits = _f32dot(u2, cent_ref[...])
    mx = jnp.max(logits, axis=-1, keepdims=True)
    eg = jnp.exp(logits - mx)
    gsum = jnp.sum(eg, axis=-1, keepdims=True)
    iot = lax.broadcasted_iota(jnp.int32, logits.shape, 1)
    top = jnp.min(jnp.where(logits == mx, iot, _E), axis=-1, keepdims=True)
    tgate = jnp.max(jnp.where(iot == top, eg, 0.0), axis=-1,
                    keepdims=True) / gsum
    ys = _f32dot(jax.nn.gelu(_f32dot(u2, ws1_ref[...]) + bs1_ref[...]),
                 ws2_ref[...]) + bs2_ref[...]
    u2_ref[...] = u2
    hp_ref[...] = h + ys
    eid_ref[...] = top[None]
    gate_ref[...] = tgate[None]


def _moe_route(hf, att, bp, cent_t):
    outs = (jax.ShapeDtypeStruct((_T, _D), jnp.float32),
            jax.ShapeDtypeStruct((_T, _D), jnp.float32),
            jax.ShapeDtypeStruct((_NT, _TILE, 1), jnp.int32),
            jax.ShapeDtypeStruct((_NT, _TILE, 1), jnp.float32))
    return pl.pallas_call(
        _route_body,
        grid=(_NT,),
        in_specs=[
            pl.BlockSpec((_TILE, _D), lambda i: (i, 0)),
            pl.BlockSpec((_TILE, _D), lambda i: (i, 0)),
            _full_spec((_D, _D)), _full_spec((1, _D)),
            _full_spec((1, _D)), _full_spec((1, _D)), _full_spec((_D, _E)),
            _full_spec((_D, _FF)), _full_spec((1, _FF)),
            _full_spec((_FF, _D)), _full_spec((1, _D)),
        ],
        out_specs=[
            pl.BlockSpec((_TILE, _D), lambda i: (i, 0)),
            pl.BlockSpec((_TILE, _D), lambda i: (i, 0)),
            pl.BlockSpec((1, _TILE, 1), lambda i: (i, 0, 0)),
            pl.BlockSpec((1, _TILE, 1), lambda i: (i, 0, 0)),
        ],
        out_shape=outs,
    )(hf, att, bp['wo'], bp['bo'].reshape(1, _D),
      bp['ln2_g'].reshape(1, _D), bp['ln2_b'].reshape(1, _D), cent_t,
      bp['ws1'], bp['bs1'].reshape(1, _FF), bp['ws2'],
      bp['bs2'].reshape(1, _D))


def _sc_mesh():
    return plsc.VectorSubcoreMesh(core_axis_name="c", subcore_axis_name="s")


def _wid():
    return lax.axis_index("s") * _NC + lax.axis_index("c")


def _pos_a_body(eid_ref, pie_ref, starts_ref, run_ref):
    i = pl.program_id(0)

    @pl.when(i == 0)
    def _():
        run_ref[...] = jnp.zeros((1, 8), jnp.float32)

    e2 = eid_ref[0]
    onehot = (e2 == lax.broadcasted_iota(jnp.int32, (_TILE, 8), 1)
              ).astype(jnp.float32)
    ri = lax.broadcasted_iota(jnp.int32, (_TILE, _TILE), 0)
    ci = lax.broadcasted_iota(jnp.int32, (_TILE, _TILE), 1)
    tril = (ri > ci).astype(jnp.float32)
    rank = lax.dot_general(tril, onehot, (((1,), (0,)), ((), ())),
                           precision=lax.Precision.HIGHEST,
                           preferred_element_type=jnp.float32)
    pie_ref[...] = jnp.sum(onehot * (rank + run_ref[...]), axis=1,
                           keepdims=True)
    run_ref[...] = run_ref[...] + jnp.sum(onehot, axis=0, keepdims=True)

    @pl.when(i == _NT - 1)
    def _():
        tot = run_ref[...]
        cu = lax.broadcasted_iota(jnp.int32, (8, 8), 0)
        cv = lax.broadcasted_iota(jnp.int32, (8, 8), 1)
        sm = (cu < cv).astype(jnp.float32)
        starts_ref[...] = lax.dot_general(
            tot, sm, (((1,), (0,)), ((), ())),
            precision=lax.Precision.HIGHEST,
            preferred_element_type=jnp.float32)

def _pos_a(eid):
    return pl.pallas_call(
        _pos_a_body,
        grid=(_NT,),
        in_specs=[pl.BlockSpec((1, _TILE, 1), lambda i: (i, 0, 0))],
        out_specs=[pl.BlockSpec((_TILE, 1), lambda i: (i, 0)),
                   pl.BlockSpec((1, 8), lambda i: (0, 0))],
        out_shape=(jax.ShapeDtypeStruct((_T, 1), jnp.float32),
                   jax.ShapeDtypeStruct((1, 8), jnp.float32)),
        scratch_shapes=[pltpu.VMEM((1, 8), jnp.float32)],
    )(eid)


def _pos_b_body(eid_ref, pie_ref, starts_ref, pos_ref, sti_ref):
    e2 = eid_ref[0]
    onehot = (e2 == lax.broadcasted_iota(jnp.int32, (_TILE, 8), 1)
              ).astype(jnp.float32)
    st = jnp.sum(onehot * starts_ref[...], axis=1, keepdims=True)
    pos_ref[...] = (pie_ref[...] + st).astype(jnp.int32)

    @pl.when(pl.program_id(0) == 0)
    def _():
        sti_ref[...] = starts_ref[...].astype(jnp.int32)


def _pos_b(eid, pie, starts_f):
    return pl.pallas_call(
        _pos_b_body,
        grid=(_NT,),
        in_specs=[pl.BlockSpec((1, _TILE, 1), lambda i: (i, 0, 0)),
                  pl.BlockSpec((_TILE, 1), lambda i: (i, 0)),
                  _full_spec((1, 8))],
        out_specs=[pl.BlockSpec((_TILE, 1), lambda i: (i, 0)),
                   pl.BlockSpec((1, 8), lambda i: (0, 0))],
        out_shape=(jax.ShapeDtypeStruct((_T, 1), jnp.int32),
                   jax.ShapeDtypeStruct((1, 8), jnp.int32)),
    )(eid, pie, starts_f)


def _sc_scatter(pos3, u2):
    @functools.partial(
        pl.kernel, mesh=_sc_mesh(),
        out_type=jax.ShapeDtypeStruct((_T, _D), jnp.float32),
        scratch_types=[pltpu.VMEM((_NCH, _CC), jnp.int32)]
        + [pltpu.VMEM((_CC, _D), jnp.float32)] * 3
        + [pltpu.SemaphoreType.DMA] * 6,
    )
    def k(pos_hbm, u2_hbm, u2s_hbm, pos_v, r0, r1, r2,
          l0, l1, l2, s0, s1, s2):
        wid = _wid()
        base = wid * _CHUNK
        pltpu.sync_copy(pos_hbm.at[wid], pos_v)
        rows = (r0, r1, r2)
        lsem = (l0, l1, l2)
        ssem = (s0, s1, s2)
        ld, st = {}, {}
        for c in range(min(2, _NCH)):
            ld[c] = pltpu.async_copy(
                u2_hbm.at[pl.ds(base + c * _CC, _CC)], rows[c % 3],
                lsem[c % 3])
        for c in range(_NCH):
            if c + 2 < _NCH:
                if c - 1 >= 0:
                    st[c - 1].wait()
                ld[c + 2] = pltpu.async_copy(
                    u2_hbm.at[pl.ds(base + (c + 2) * _CC, _CC)],
                    rows[(c + 2) % 3], lsem[(c + 2) % 3])
            ld[c].wait()
            st[c] = pltpu.async_copy(rows[c % 3], u2s_hbm.at[pos_v.at[c]],
                                     ssem[c % 3])
        for c in range(max(0, _NCH - 3), _NCH):
            st[c].wait()

    return k(pos3, u2)


def _sc_unsort(pos3, ys):
    @functools.partial(
        pl.kernel, mesh=_sc_mesh(),
        out_type=jax.ShapeDtypeStruct((_T, _D), jnp.float32),
        scratch_types=[pltpu.VMEM((_NCH, _CC), jnp.int32)]
        + [pltpu.VMEM((_CC, _D), jnp.float32)] * 3
        + [pltpu.SemaphoreType.DMA] * 6,
    )
    def k(pos_hbm, ys_hbm, yu_hbm, pos_v, r0, r1, r2,
          g0, g1, g2, s0, s1, s2):
        wid = _wid()
        base = wid * _CHUNK
        pltpu.sync_copy(pos_hbm.at[wid], pos_v)
        rows = (r0, r1, r2)
        gsem = (g0, g1, g2)
        ssem = (s0, s1, s2)
        gd, st = {}, {}
        for c in range(min(2, _NCH)):
            gd[c] = pltpu.async_copy(ys_hbm.at[pos_v.at[c]], rows[c % 3],
                                     gsem[c % 3])
        for c in range(_NCH):
            if c + 2 < _NCH:
                if c - 1 >= 0:
                    st[c - 1].wait()
                gd[c + 2] = pltpu.async_copy(
                    ys_hbm.at[pos_v.at[c + 2]], rows[(c + 2) % 3],
                    gsem[(c + 2) % 3])
            gd[c].wait()
            st[c] = pltpu.async_copy(
                rows[c % 3], yu_hbm.at[pl.ds(base + c * _CC, _CC)],
                ssem[c % 3])
        for c in range(max(0, _NCH - 3), _NCH):
            st[c].wait()

    return k(pos3, ys)


def _group_body(starts_ref, u_ref, w1_ref, b1_ref, w2_ref, b2e_ref, o_ref,
                acc_ref):
    t0 = pl.program_id(0) * _TILE
    u = u_ref[...]
    b1 = b1_ref[...]
    b2e = b2e_ref[...]
    rowid = t0 + lax.broadcasted_iota(jnp.int32, (_TILE, 1), 0)
    acc_ref[...] = jnp.zeros((_TILE, _D), jnp.float32)
    for e in range(_E):
        lo = jnp.maximum(starts_ref[e], t0)
        hi = jnp.minimum(starts_ref[e + 1], t0 + _TILE)

        @pl.when(lo < hi)
        def _(e=e, lo=lo, hi=hi):
            t1 = jax.nn.gelu(_f32dot(u, w1_ref[e]) + b1[e][None])
            t2 = _f32dot(t1, w2_ref[e]) + b2e[e][None]
            m = (rowid >= lo) & (rowid < hi)
            acc_ref[...] += jnp.where(m, t2, 0.0)

    o_ref[...] = acc_ref[...]


def _moe_group(u2s, starts, bp):
    grid_spec = pltpu.PrefetchScalarGridSpec(
        num_scalar_prefetch=1,
        grid=(_NT,),
        in_specs=[
            pl.BlockSpec((_TILE, _D), lambda i, s: (i, 0)),
            pl.BlockSpec((_E, _D, _FF), lambda i, s: (0, 0, 0)),
            pl.BlockSpec((_E, _FF), lambda i, s: (0, 0)),
            pl.BlockSpec((_E, _FF, _D), lambda i, s: (0, 0, 0)),
            pl.BlockSpec((_E, _D), lambda i, s: (0, 0)),
        ],
        out_specs=pl.BlockSpec((_TILE, _D), lambda i, s: (i, 0)),
        scratch_shapes=[pltpu.VMEM((_TILE, _D), jnp.float32)],
    )
    return pl.pallas_call(
        _group_body,
        grid_spec=grid_spec,
        out_shape=jax.ShapeDtypeStruct((_T, _D), jnp.float32),
    )(starts, u2s, bp['w1'], bp['b1'], bp['w2'], bp['b2'])


def _final_body(hc_ref, a2_ref, wo_ref, bo_ref, g2_ref,
                b2_ref, cent_ref, w1_ref, b1_ref, w2_ref, b2e_ref, ws1_ref,
                bs1_ref, ws2_ref, bs2_ref, fcw_ref, fcb_ref, o_ref):
    c = hc_ref[...] + _f32dot(a2_ref[...], wo_ref[...]) + bo_ref[...]
    o = _moe_math(c, g2_ref[...], b2_ref[...], cent_ref[...], w1_ref[...],
                  b1_ref[...], w2_ref[...], b2e_ref[...], ws1_ref[...],
                  bs1_ref[...], ws2_ref[...], bs2_ref[...])
    o_ref[...] = _f32dot(o, fcw_ref[...]) + fcb_ref[...]


def _final(hcls, att2, bp, cent_t, p):
    bb = 512
    return pl.pallas_call(
        _final_body,
        grid=(_B // bb,),
        in_specs=[
            pl.BlockSpec((bb, _D), lambda i: (i, 0)),
            pl.BlockSpec((bb, _D), lambda i: (i, 0)),
            _full_spec((_D, _D)), _full_spec((1, _D)),
            _full_spec((1, _D)), _full_spec((1, _D)), _full_spec((_D, _E)),
            _full_spec((_E, _D, _FF)), _full_spec((_E, _FF)),
            _full_spec((_E, _FF, _D)), _full_spec((_E, _D)),
            _full_spec((_D, _FF)), _full_spec((1, _FF)),
            _full_spec((_FF, _D)), _full_spec((1, _D)),
            _full_spec((_D, 3)), _full_spec((1, 3)),
        ],
        out_specs=pl.BlockSpec((bb, 3), lambda i: (i, 0)),
        out_shape=jax.ShapeDtypeStruct((_B, 3), jnp.float32),
    )(hcls, att2, bp['wo'], bp['bo'].reshape(1, _D),
      bp['ln2_g'].reshape(1, _D), bp['ln2_b'].reshape(1, _D), cent_t,
      bp['w1'], bp['b1'], bp['w2'], bp['b2'],
      bp['ws1'], bp['bs1'].reshape(1, _FF), bp['ws2'],
      bp['bs2'].reshape(1, _D), p['fc_w'], p['fc_b'].reshape(1, 3))


def kernel(x, params):
    p = params
    b0, b1 = p['blocks']
    cent_t = p['centroids'].T

    h0, qkv1 = _embed_qkv(x, p, b0)
    att1 = _attn1(qkv1.reshape(_B, _S, 3 * _D))

    u2, hplus, eid, gate = _moe_route(h0.reshape(_T, _D),
                                      att1.reshape(_T, _D), b0, cent_t)
    pie, starts_f = _pos_a(eid)
    pos2d, starts_i = _pos_b(eid, pie, starts_f)
    pos3 = pos2d.reshape(_NW, _NCH, _CC)
    starts = starts_i.reshape(8)
    u2s = _sc_scatter(pos3, u2)
    ys = _moe_group(u2s, starts, b0)
    yu = _sc_unsort(pos3, ys)

    att2, h1cls = _combine_attn2(hplus, yu, gate.reshape(_T, 1), b1)
    return _final(h1cls, att2, b1, cent_t, p)
